# baseline (speedup 1.0000x reference)
import functools

import jax
import jax.numpy as jnp
from jax import lax
from jax.experimental import pallas as pl
from jax.experimental.pallas import tpu as pltpu
from jax.experimental.pallas import tpu_sc as plsc

N = 10000
D = 128
E = 320000
NC = 2
NS = 16
NW = NC * NS
CHUNK = 128
C = 80
E_PAD = NW * C * CHUNK
N_PAD = 10240
ROWS_PT = N_PAD // NS
NB = 2

_mesh = plsc.VectorSubcoreMesh(core_axis_name="c", subcore_axis_name="s")
_f32 = jnp.float32


@functools.partial(
    pl.kernel,
    out_type=jax.ShapeDtypeStruct((NC, N_PAD), _f32),
    mesh=_mesh,
    scratch_types=[
        pltpu.VMEM((C, CHUNK), jnp.int32),
        pltpu.VMEM((640,), _f32),
        pltpu.VMEM((CHUNK,), _f32),
        pltpu.VMEM_SHARED((N_PAD,), _f32),
    ],
)
def _deg_kernel(dst_hbm, out_hbm, dst_v, zbuf, ones_v, deg_sh):
    c = lax.axis_index("c")
    s = lax.axis_index("s")
    wid = s * NC + c
    for k in range(40):
        zbuf[pl.ds(k * 16, 16)] = jnp.zeros((16,), _f32)
    for k in range(8):
        ones_v[pl.ds(k * 16, 16)] = jnp.ones((16,), _f32)
    pltpu.sync_copy(dst_hbm.at[wid], dst_v)
    pltpu.sync_copy(zbuf.at[pl.ds(0, ROWS_PT)], deg_sh.at[pl.ds(s * ROWS_PT, ROWS_PT)])
    plsc.subcore_barrier()

    def body(j, carry):
        pltpu.sync_copy(ones_v, deg_sh.at[dst_v.at[j]], add=True)
        return carry

    lax.fori_loop(0, C, body, 0)
    plsc.subcore_barrier()
    pltpu.sync_copy(
        deg_sh.at[pl.ds(s * ROWS_PT, ROWS_PT)],
        out_hbm.at[c].at[pl.ds(s * ROWS_PT, ROWS_PT)],
    )


EPT = C * CHUNK
NBKT = 32
BKT_W = 320
NV = EPT // 16
_lp_params = pltpu.CompilerParams(needs_layout_passes=False)


def _bucket_slot(sv, lane):
    return lax.shift_left(sv // BKT_W, 4) + lane


@functools.partial(
    pl.kernel,
    out_type=jax.ShapeDtypeStruct((NW, NBKT * 16), jnp.int32),
    mesh=_mesh,
    scratch_types=[
        pltpu.VMEM((EPT,), jnp.int32),
        pltpu.VMEM((NBKT * 16,), jnp.int32),
    ],
    compiler_params=_lp_params,
)
def _count_kernel(src_hbm, ocnt_hbm, src_v, cnt_v):
    c = lax.axis_index("c")
    s = lax.axis_index("s")
    wid = s * NC + c
    pltpu.sync_copy(src_hbm.at[wid], src_v)
    lane = lax.iota(jnp.int32, 16)
    zero = jnp.zeros((16,), jnp.int32)
    for b in range(NBKT):
        cnt_v[pl.ds(b * 16, 16)] = zero

    def count_step(t, carry):
        sv = src_v[pl.ds(t * 16, 16)]
        slot = _bucket_slot(sv, lane)
        cur = plsc.load_gather(cnt_v, [slot])
        plsc.store_scatter(cnt_v, [slot], cur + 1)
        return carry

    lax.fori_loop(0, NV, count_step, 0)
    pltpu.sync_copy(cnt_v, ocnt_hbm.at[wid])


@functools.partial(
    pl.kernel,
    out_type=[
        jax.ShapeDtypeStruct((E_PAD,), jnp.int32),
        jax.ShapeDtypeStruct((E_PAD,), jnp.int32),
    ],
    mesh=_mesh,
    scratch_types=[
        pltpu.VMEM((EPT,), jnp.int32),
        pltpu.VMEM((EPT,), jnp.int32),
        pltpu.VMEM((NW, NBKT * 16), jnp.int32),
        pltpu.VMEM((NBKT * 16,), jnp.int32),
        pltpu.VMEM((EPT // CHUNK, CHUNK), jnp.int32),
        pltpu.VMEM((32,), jnp.int32),
    ],
    compiler_params=_lp_params,
)
def _place_kernel(src_hbm, dst_hbm, cnt_hbm, osrc_hbm, odst_hbm,
                  src_v, dst_v, allc_v, cur_v, pos_v, shf):
    c = lax.axis_index("c")
    s = lax.axis_index("s")
    wid = s * NC + c
    pltpu.sync_copy(src_hbm.at[wid], src_v)
    pltpu.sync_copy(dst_hbm.at[wid], dst_v)
    pltpu.sync_copy(cnt_hbm, allc_v)

    lane = lax.iota(jnp.int32, 16)
    zero = jnp.zeros((16,), jnp.int32)
    shf[pl.ds(0, 16)] = zero

    def lane_sum(x):
        for k in (1, 2, 4, 8):
            shf[pl.ds(8, 16)] = x
            x = x + shf[pl.ds(8 - k, 16)]
        return x

    gbase = jnp.int32(0)
    for b in range(NBKT):
        pre = zero
        allv = zero
        mine = zero
        for i in range(NW):
            v = allc_v[i, pl.ds(b * 16, 16)]
            pre = jnp.where(i < wid, pre + v, pre)
            mine = jnp.where(i == wid, v, mine)
            allv = allv + v
        pre_tiles = lane_sum(pre)[15]
        mine_in = lane_sum(mine)
        shf[pl.ds(8, 16)] = mine_in
        mine_excl = shf[pl.ds(7, 16)]
        cur_v[pl.ds(b * 16, 16)] = gbase + pre_tiles + mine_excl
        gbase = gbase + lane_sum(allv)[15]

    def place_step(t, carry):
        sv = src_v[pl.ds(t * 16, 16)]
        slot = _bucket_slot(sv, lane)
        pos = plsc.load_gather(cur_v, [slot])
        plsc.store_scatter(cur_v, [slot], pos + 1)
        pos_v[t // 8, pl.ds((t % 8) * 16, 16)] = pos
        return carry

    lax.fori_loop(0, NV, place_step, 0)

    def scat_step(j, carry):
        pltpu.sync_copy(src_v.at[pl.ds(j * CHUNK, CHUNK)], osrc_hbm.at[pos_v.at[j]])
        pltpu.sync_copy(dst_v.at[pl.ds(j * CHUNK, CHUNK)], odst_hbm.at[pos_v.at[j]])
        return carry

    lax.fori_loop(0, EPT // CHUNK, scat_step, 0)


@functools.partial(
    pl.kernel,
    out_type=jax.ShapeDtypeStruct((NC, N_PAD, D), _f32),
    mesh=_mesh,
    scratch_types=[
        pltpu.VMEM((C // 2, CHUNK), jnp.int32),
        pltpu.VMEM((C // 2, CHUNK), jnp.int32),
        pltpu.VMEM((NB, CHUNK, D), _f32),
        pltpu.VMEM_SHARED((N_PAD, D), _f32),
        pltpu.SemaphoreType.DMA,
        pltpu.SemaphoreType.DMA,
    ],
)
def _agg_kernel(h_hbm, src_hbm, dst_hbm, out_hbm,
                src_v, dst_v, gbuf, acc_sh, sem0, sem1):
    c = lax.axis_index("c")
    s = lax.axis_index("s")
    wid = s * NC + c
    sems = [sem0, sem1]
    HC = C // 2

    def zrow(i, carry):
        for k in range(8):
            gbuf[0, i, pl.ds(k * 16, 16)] = jnp.zeros((16,), _f32)
        return carry

    lax.fori_loop(0, CHUNK, zrow, 0)
    for k in range(ROWS_PT // CHUNK):
        pltpu.sync_copy(gbuf.at[0], acc_sh.at[pl.ds(s * ROWS_PT + k * CHUNK, CHUNK)])
    plsc.subcore_barrier()

    for h in range(2):
        pltpu.sync_copy(src_hbm.at[wid].at[pl.ds(h * HC, HC)], src_v)
        pltpu.sync_copy(dst_hbm.at[wid].at[pl.ds(h * HC, HC)], dst_v)
        for b in range(NB):
            pltpu.async_copy(h_hbm.at[src_v.at[b]], gbuf.at[b], sems[b])

        def body(g, carry):
            base = g * NB
            for b in range(NB):
                j = base + b
                pltpu.make_async_copy(h_hbm.at[src_v.at[j]], gbuf.at[b], sems[b]).wait()
                pltpu.sync_copy(gbuf.at[b], acc_sh.at[dst_v.at[j]], add=True)
                nxt = j + NB

                @pl.when(nxt < HC)
                def _issue():
                    pltpu.async_copy(h_hbm.at[src_v.at[nxt]], gbuf.at[b], sems[b])

            return carry

        lax.fori_loop(0, HC // NB, body, 0)
    plsc.subcore_barrier()
    pltpu.sync_copy(
        acc_sh.at[pl.ds(s * ROWS_PT, ROWS_PT)],
        out_hbm.at[c].at[pl.ds(s * ROWS_PT, ROWS_PT)],
    )


_BR = 1000
_G = N // _BR


def _tc1_body(d_ref0, d_ref1, x_ref, w1_ref, ht_ref, dinv_ref):
    deg = d_ref0[0] + d_ref1[0] + 1.0
    dinv = lax.rsqrt(deg)
    h = jnp.dot(x_ref[...], w1_ref[...], preferred_element_type=_f32)
    ht_ref[...] = h * dinv
    dinv_ref[...] = dinv


def _tc1(degp, x, w1):
    return pl.pallas_call(
        _tc1_body,
        grid=(_G,),
        in_specs=[
            pl.BlockSpec((1, _BR, 1), lambda i: (0, i, 0)),
            pl.BlockSpec((1, _BR, 1), lambda i: (1, i, 0)),
            pl.BlockSpec((_BR, D), lambda i: (i, 0)),
            pl.BlockSpec((D, D), lambda i: (0, 0)),
        ],
        out_specs=[
            pl.BlockSpec((_BR, D), lambda i: (i, 0)),
            pl.BlockSpec((_BR, 1), lambda i: (i, 0)),
        ],
        out_shape=[
            jax.ShapeDtypeStruct((N, D), _f32),
            jax.ShapeDtypeStruct((N, 1), _f32),
        ],
    )(degp, degp, x, w1)


def _tc2_body(a_ref0, a_ref1, ht_ref, dinv_ref, b1_ref, w2_ref, ht2_ref):
    pre = (a_ref0[0] + a_ref1[0] + ht_ref[...]) * dinv_ref[...] + b1_ref[...]
    act = jnp.tanh(pre)
    h2 = jnp.dot(act, w2_ref[...], preferred_element_type=_f32)
    ht2_ref[...] = h2 * dinv_ref[...]


def _tc2(agg, ht, dinv, b1, w2):
    return pl.pallas_call(
        _tc2_body,
        grid=(_G,),
        in_specs=[
            pl.BlockSpec((1, _BR, D), lambda i: (0, i, 0)),
            pl.BlockSpec((1, _BR, D), lambda i: (1, i, 0)),
            pl.BlockSpec((_BR, D), lambda i: (i, 0)),
            pl.BlockSpec((_BR, 1), lambda i: (i, 0)),
            pl.BlockSpec((1, D), lambda i: (0, 0)),
            pl.BlockSpec((D, D), lambda i: (0, 0)),
        ],
        out_specs=pl.BlockSpec((_BR, D), lambda i: (i, 0)),
        out_shape=jax.ShapeDtypeStruct((N, D), _f32),
    )(agg, agg, ht, dinv, b1, w2)


def _tc3_body(a_ref0, a_ref1, ht2_ref, dinv_ref, b2_ref,
              wf1_ref, bf1_ref, wf2_ref, bf2_ref, out_ref):
    pre = (a_ref0[0] + a_ref1[0] + ht2_ref[...]) * dinv_ref[...] + b2_ref[...]
    act = jnp.tanh(pre)
    h3 = jnp.tanh(jnp.dot(act, wf1_ref[...], preferred_element_type=_f32)
                  + bf1_ref[...])
    out_ref[...] = jnp.dot(h3, wf2_ref[...], preferred_element_type=_f32) + bf2_ref[...]


def _tc3(agg, ht2, dinv, b2, wf1, bf1, wf2, bf2):
    return pl.pallas_call(
        _tc3_body,
        grid=(_G,),
        in_specs=[
            pl.BlockSpec((1, _BR, D), lambda i: (0, i, 0)),
            pl.BlockSpec((1, _BR, D), lambda i: (1, i, 0)),
            pl.BlockSpec((_BR, D), lambda i: (i, 0)),
            pl.BlockSpec((_BR, 1), lambda i: (i, 0)),
            pl.BlockSpec((1, D), lambda i: (0, 0)),
            pl.BlockSpec((D, 64), lambda i: (0, 0)),
            pl.BlockSpec((1, 64), lambda i: (0, 0)),
            pl.BlockSpec((64, 1), lambda i: (0, 0)),
            pl.BlockSpec((1, 1), lambda i: (0, 0)),
        ],
        out_specs=pl.BlockSpec((_BR, 1), lambda i: (i, 0)),
        out_shape=jax.ShapeDtypeStruct((N, 1), _f32),
    )(agg, agg, ht2, dinv, b2, wf1, bf1, wf2, bf2)


def kernel(x, edge_index, W1, b1, W2, b2, Wf1, bf1, Wf2, bf2):
    src = edge_index[0].astype(jnp.int32)
    dst = edge_index[1].astype(jnp.int32)
    pad = E_PAD - E
    ar = jnp.arange(pad, dtype=jnp.int32)
    dump = N + (ar % (N_PAD - N))
    src_p = jnp.concatenate([src, ar % N])
    dst_p = jnp.concatenate([dst, dump])
    src2 = src_p.reshape(NW, EPT)
    dst2 = dst_p.reshape(NW, EPT)
    dst3 = dst_p.reshape(NW, C, CHUNK)

    cnts = _count_kernel(src2)
    srcg, dstg = _place_kernel(src2, dst2, cnts)
    srcg = srcg.reshape(NW, C, CHUNK)
    dstg = dstg.reshape(NW, C, CHUNK)

    degp = _deg_kernel(dst3)[:, :N].reshape(NC, N, 1)
    ht1, dinv = _tc1(degp, x, W1)
    agg1 = _agg_kernel(ht1, srcg, dstg)
    ht2 = _tc2(agg1, ht1, dinv, b1.reshape(1, D), W2)
    agg2 = _agg_kernel(ht2, srcg, dstg)
    return _tc3(agg2, ht2, dinv, b2.reshape(1, D),
                Wf1, bf1.reshape(1, 64), Wf2, bf2.reshape(1, 1))


# speedup vs baseline: 2.6520x; 2.6520x over previous
import functools

import jax
import jax.numpy as jnp
from jax import lax
from jax.experimental import pallas as pl
from jax.experimental.pallas import tpu as pltpu
from jax.experimental.pallas import tpu_sc as plsc

N = 10000
D = 128
E = 320000
NC = 2
NS = 16
NW = NC * NS
CHUNK = 128
C = 80
E_PAD = NW * C * CHUNK
N_PAD = 10240
ROWS_PT = N_PAD // NS
NB = 2

_mesh = plsc.VectorSubcoreMesh(core_axis_name="c", subcore_axis_name="s")
_f32 = jnp.float32


@functools.partial(
    pl.kernel,
    out_type=jax.ShapeDtypeStruct((NC, N_PAD), _f32),
    mesh=_mesh,
    scratch_types=[
        pltpu.VMEM((C, CHUNK), jnp.int32),
        pltpu.VMEM((640,), _f32),
        pltpu.VMEM((CHUNK,), _f32),
        pltpu.VMEM_SHARED((N_PAD,), _f32),
    ],
)
def _deg_kernel(dst_hbm, out_hbm, dst_v, zbuf, ones_v, deg_sh):
    c = lax.axis_index("c")
    s = lax.axis_index("s")
    wid = s * NC + c
    for k in range(40):
        zbuf[pl.ds(k * 16, 16)] = jnp.zeros((16,), _f32)
    for k in range(8):
        ones_v[pl.ds(k * 16, 16)] = jnp.ones((16,), _f32)
    pltpu.sync_copy(dst_hbm.at[wid], dst_v)
    pltpu.sync_copy(zbuf.at[pl.ds(0, ROWS_PT)], deg_sh.at[pl.ds(s * ROWS_PT, ROWS_PT)])
    plsc.subcore_barrier()

    def body(j, carry):
        pltpu.sync_copy(ones_v, deg_sh.at[dst_v.at[j]], add=True)
        return carry

    lax.fori_loop(0, C, body, 0)
    plsc.subcore_barrier()
    pltpu.sync_copy(
        deg_sh.at[pl.ds(s * ROWS_PT, ROWS_PT)],
        out_hbm.at[c].at[pl.ds(s * ROWS_PT, ROWS_PT)],
    )


EPT = C * CHUNK
NBKT = 32
BKT_W = 320
NV = EPT // 16
_lp_params = pltpu.CompilerParams(needs_layout_passes=False)


def _bucket_slot(sv, lane):
    return lax.shift_left(sv // BKT_W, 4) + lane


@functools.partial(
    pl.kernel,
    out_type=[
        jax.ShapeDtypeStruct((NC, NS * EPT), jnp.int32),
        jax.ShapeDtypeStruct((NC, NS * EPT), jnp.int32),
    ],
    mesh=_mesh,
    scratch_types=[
        pltpu.VMEM((EPT,), jnp.int32),
        pltpu.VMEM((EPT,), jnp.int32),
        pltpu.VMEM((NS, NBKT * 16), jnp.int32),
        pltpu.VMEM((NBKT * 16,), jnp.int32),
        pltpu.VMEM((EPT // CHUNK, CHUNK), jnp.int32),
        pltpu.VMEM((32,), jnp.int32),
        pltpu.VMEM_SHARED((NS, NBKT * 16), jnp.int32),
        pltpu.VMEM_SHARED((NS * EPT,), jnp.int32),
        pltpu.VMEM_SHARED((NS * EPT,), jnp.int32),
    ],
    compiler_params=_lp_params,
)
def _sort_kernel(src_hbm, dst_hbm, osrc_hbm, odst_hbm,
                 src_v, dst_v, allc_v, cur_v, pos_v, shf,
                 cnt_sh, ssrc_sh, sdst_sh):
    c = lax.axis_index("c")
    s = lax.axis_index("s")
    erow = c * NS + s
    pltpu.sync_copy(src_hbm.at[erow], src_v)
    pltpu.sync_copy(dst_hbm.at[erow], dst_v)

    lane = lax.iota(jnp.int32, 16)
    zero = jnp.zeros((16,), jnp.int32)
    for b in range(NBKT):
        cur_v[pl.ds(b * 16, 16)] = zero
    shf[pl.ds(0, 16)] = zero

    def count_step(t, carry):
        sv = src_v[pl.ds(t * 16, 16)]
        slot = _bucket_slot(sv, lane)
        cur = plsc.load_gather(cur_v, [slot])
        plsc.store_scatter(cur_v, [slot], cur + 1)
        return carry

    lax.fori_loop(0, NV, count_step, 0)
    pltpu.sync_copy(cur_v, cnt_sh.at[s])
    plsc.subcore_barrier()
    pltpu.sync_copy(cnt_sh, allc_v)

    def lane_sum(x):
        for k in (1, 2, 4, 8):
            shf[pl.ds(8, 16)] = x
            x = x + shf[pl.ds(8 - k, 16)]
        return x

    gbase = jnp.int32(0)
    for b in range(NBKT):
        pre = zero
        allv = zero
        mine = zero
        for i in range(NS):
            v = allc_v[i, pl.ds(b * 16, 16)]
            pre = jnp.where(i < s, pre + v, pre)
            mine = jnp.where(i == s, v, mine)
            allv = allv + v
        pre_tiles = lane_sum(pre)[15]
        mine_in = lane_sum(mine)
        shf[pl.ds(8, 16)] = mine_in
        mine_excl = shf[pl.ds(7, 16)]
        cur_v[pl.ds(b * 16, 16)] = gbase + pre_tiles + mine_excl
        gbase = gbase + lane_sum(allv)[15]

    def place_step(t, carry):
        sv = src_v[pl.ds(t * 16, 16)]
        slot = _bucket_slot(sv, lane)
        pos = plsc.load_gather(cur_v, [slot])
        plsc.store_scatter(cur_v, [slot], pos + 1)
        pos_v[t // 8, pl.ds((t % 8) * 16, 16)] = pos
        return carry

    lax.fori_loop(0, NV, place_step, 0)

    def scat_step(j, carry):
        pltpu.sync_copy(src_v.at[pl.ds(j * CHUNK, CHUNK)], ssrc_sh.at[pos_v.at[j]])
        pltpu.sync_copy(dst_v.at[pl.ds(j * CHUNK, CHUNK)], sdst_sh.at[pos_v.at[j]])
        return carry

    lax.fori_loop(0, EPT // CHUNK, scat_step, 0)
    plsc.subcore_barrier()
    pltpu.sync_copy(ssrc_sh.at[pl.ds(s * EPT, EPT)], osrc_hbm.at[c].at[pl.ds(s * EPT, EPT)])
    pltpu.sync_copy(sdst_sh.at[pl.ds(s * EPT, EPT)], odst_hbm.at[c].at[pl.ds(s * EPT, EPT)])


@functools.partial(
    pl.kernel,
    out_type=jax.ShapeDtypeStruct((NC, N_PAD, D), _f32),
    mesh=_mesh,
    scratch_types=[
        pltpu.VMEM((C // 2, CHUNK), jnp.int32),
        pltpu.VMEM((C // 2, CHUNK), jnp.int32),
        pltpu.VMEM((NB, CHUNK, D), _f32),
        pltpu.VMEM_SHARED((N_PAD, D), _f32),
        pltpu.SemaphoreType.DMA,
        pltpu.SemaphoreType.DMA,
    ],
)
def _agg_kernel(h_hbm, src_hbm, dst_hbm, out_hbm,
                src_v, dst_v, gbuf, acc_sh, sem0, sem1):
    c = lax.axis_index("c")
    s = lax.axis_index("s")
    wid = c * NS + s
    sems = [sem0, sem1]
    HC = C // 2

    def zrow(i, carry):
        for k in range(8):
            gbuf[0, i, pl.ds(k * 16, 16)] = jnp.zeros((16,), _f32)
        return carry

    lax.fori_loop(0, CHUNK, zrow, 0)
    for k in range(ROWS_PT // CHUNK):
        pltpu.sync_copy(gbuf.at[0], acc_sh.at[pl.ds(s * ROWS_PT + k * CHUNK, CHUNK)])
    plsc.subcore_barrier()

    for h in range(2):
        pltpu.sync_copy(src_hbm.at[wid].at[pl.ds(h * HC, HC)], src_v)
        pltpu.sync_copy(dst_hbm.at[wid].at[pl.ds(h * HC, HC)], dst_v)
        for b in range(NB):
            pltpu.async_copy(h_hbm.at[src_v.at[b]], gbuf.at[b], sems[b])

        def body(g, carry):
            base = g * NB
            for b in range(NB):
                j = base + b
                pltpu.make_async_copy(h_hbm.at[src_v.at[j]], gbuf.at[b], sems[b]).wait()
                pltpu.sync_copy(gbuf.at[b], acc_sh.at[dst_v.at[j]], add=True)
                nxt = j + NB

                @pl.when(nxt < HC)
                def _issue():
                    pltpu.async_copy(h_hbm.at[src_v.at[nxt]], gbuf.at[b], sems[b])

            return carry

        lax.fori_loop(0, HC // NB, body, 0)
    plsc.subcore_barrier()
    pltpu.sync_copy(
        acc_sh.at[pl.ds(s * ROWS_PT, ROWS_PT)],
        out_hbm.at[c].at[pl.ds(s * ROWS_PT, ROWS_PT)],
    )


_BR = 1000
_G = N // _BR


def _tc1_body(d_ref0, d_ref1, x_ref, w1_ref, ht_ref, dinv_ref):
    deg = d_ref0[0] + d_ref1[0] + 1.0
    dinv = lax.rsqrt(deg)
    h = jnp.dot(x_ref[...], w1_ref[...], preferred_element_type=_f32)
    ht_ref[...] = h * dinv
    dinv_ref[...] = dinv


def _tc1(degp, x, w1):
    return pl.pallas_call(
        _tc1_body,
        grid=(_G,),
        in_specs=[
            pl.BlockSpec((1, _BR, 1), lambda i: (0, i, 0)),
            pl.BlockSpec((1, _BR, 1), lambda i: (1, i, 0)),
            pl.BlockSpec((_BR, D), lambda i: (i, 0)),
            pl.BlockSpec((D, D), lambda i: (0, 0)),
        ],
        out_specs=[
            pl.BlockSpec((_BR, D), lambda i: (i, 0)),
            pl.BlockSpec((_BR, 1), lambda i: (i, 0)),
        ],
        out_shape=[
            jax.ShapeDtypeStruct((N, D), _f32),
            jax.ShapeDtypeStruct((N, 1), _f32),
        ],
    )(degp, degp, x, w1)


def _tc2_body(a_ref0, a_ref1, ht_ref, dinv_ref, b1_ref, w2_ref, ht2_ref):
    pre = (a_ref0[0] + a_ref1[0] + ht_ref[...]) * dinv_ref[...] + b1_ref[...]
    act = jnp.tanh(pre)
    h2 = jnp.dot(act, w2_ref[...], preferred_element_type=_f32)
    ht2_ref[...] = h2 * dinv_ref[...]


def _tc2(agg, ht, dinv, b1, w2):
    return pl.pallas_call(
        _tc2_body,
        grid=(_G,),
        in_specs=[
            pl.BlockSpec((1, _BR, D), lambda i: (0, i, 0)),
            pl.BlockSpec((1, _BR, D), lambda i: (1, i, 0)),
            pl.BlockSpec((_BR, D), lambda i: (i, 0)),
            pl.BlockSpec((_BR, 1), lambda i: (i, 0)),
            pl.BlockSpec((1, D), lambda i: (0, 0)),
            pl.BlockSpec((D, D), lambda i: (0, 0)),
        ],
        out_specs=pl.BlockSpec((_BR, D), lambda i: (i, 0)),
        out_shape=jax.ShapeDtypeStruct((N, D), _f32),
    )(agg, agg, ht, dinv, b1, w2)


def _tc3_body(a_ref0, a_ref1, ht2_ref, dinv_ref, b2_ref,
              wf1_ref, bf1_ref, wf2_ref, bf2_ref, out_ref):
    pre = (a_ref0[0] + a_ref1[0] + ht2_ref[...]) * dinv_ref[...] + b2_ref[...]
    act = jnp.tanh(pre)
    h3 = jnp.tanh(jnp.dot(act, wf1_ref[...], preferred_element_type=_f32)
                  + bf1_ref[...])
    out_ref[...] = jnp.dot(h3, wf2_ref[...], preferred_element_type=_f32) + bf2_ref[...]


def _tc3(agg, ht2, dinv, b2, wf1, bf1, wf2, bf2):
    return pl.pallas_call(
        _tc3_body,
        grid=(_G,),
        in_specs=[
            pl.BlockSpec((1, _BR, D), lambda i: (0, i, 0)),
            pl.BlockSpec((1, _BR, D), lambda i: (1, i, 0)),
            pl.BlockSpec((_BR, D), lambda i: (i, 0)),
            pl.BlockSpec((_BR, 1), lambda i: (i, 0)),
            pl.BlockSpec((1, D), lambda i: (0, 0)),
            pl.BlockSpec((D, 64), lambda i: (0, 0)),
            pl.BlockSpec((1, 64), lambda i: (0, 0)),
            pl.BlockSpec((64, 1), lambda i: (0, 0)),
            pl.BlockSpec((1, 1), lambda i: (0, 0)),
        ],
        out_specs=pl.BlockSpec((_BR, 1), lambda i: (i, 0)),
        out_shape=jax.ShapeDtypeStruct((N, 1), _f32),
    )(agg, agg, ht2, dinv, b2, wf1, bf1, wf2, bf2)


def kernel(x, edge_index, W1, b1, W2, b2, Wf1, bf1, Wf2, bf2):
    src = edge_index[0].astype(jnp.int32)
    dst = edge_index[1].astype(jnp.int32)
    pad = E_PAD - E
    ar = jnp.arange(pad, dtype=jnp.int32)
    dump = N + (ar % (N_PAD - N))
    src_p = jnp.concatenate([src, ar % N])
    dst_p = jnp.concatenate([dst, dump])
    src2 = src_p.reshape(NW, EPT)
    dst2 = dst_p.reshape(NW, EPT)
    dst3 = dst_p.reshape(NW, C, CHUNK)

    srcg, dstg = _sort_kernel(src2, dst2)
    srcg = srcg.reshape(NW, C, CHUNK)
    dstg = dstg.reshape(NW, C, CHUNK)

    degp = _deg_kernel(dst3)[:, :N].reshape(NC, N, 1)
    ht1, dinv = _tc1(degp, x, W1)
    agg1 = _agg_kernel(ht1, srcg, dstg)
    ht2 = _tc2(agg1, ht1, dinv, b1.reshape(1, D), W2)
    agg2 = _agg_kernel(ht2, srcg, dstg)
    return _tc3(agg2, ht2, dinv, b2.reshape(1, D),
                Wf1, bf1.reshape(1, 64), Wf2, bf2.reshape(1, 1))


# speedup vs baseline: 2.6740x; 1.0083x over previous
import functools

import jax
import jax.numpy as jnp
from jax import lax
from jax.experimental import pallas as pl
from jax.experimental.pallas import tpu as pltpu
from jax.experimental.pallas import tpu_sc as plsc

N = 10000
D = 128
E = 320000
NC = 2
NS = 16
NW = NC * NS
CHUNK = 128
C = 80
E_PAD = NW * C * CHUNK
N_PAD = 10240
ROWS_PT = N_PAD // NS
NB = 2

_mesh = plsc.VectorSubcoreMesh(core_axis_name="c", subcore_axis_name="s")
_f32 = jnp.float32


EPT = C * CHUNK
NBKT = 32
BKT_W = 320
NV = EPT // 16
_lp_params = pltpu.CompilerParams(needs_layout_passes=False)


def _bucket_slot(sv, lane):
    return lax.shift_left(sv // BKT_W, 4) + lane


@functools.partial(
    pl.kernel,
    out_type=[
        jax.ShapeDtypeStruct((NC, NS * EPT), jnp.int32),
        jax.ShapeDtypeStruct((NC, NS * EPT), jnp.int32),
        jax.ShapeDtypeStruct((NC, N_PAD), _f32),
    ],
    mesh=_mesh,
    scratch_types=[
        pltpu.VMEM((EPT,), jnp.int32),
        pltpu.VMEM((C, CHUNK), jnp.int32),
        pltpu.VMEM((NS, NBKT * 16), jnp.int32),
        pltpu.VMEM((NBKT * 16,), jnp.int32),
        pltpu.VMEM((EPT // CHUNK, CHUNK), jnp.int32),
        pltpu.VMEM((32,), jnp.int32),
        pltpu.VMEM((640,), _f32),
        pltpu.VMEM((CHUNK,), _f32),
        pltpu.VMEM_SHARED((NS, NBKT * 16), jnp.int32),
        pltpu.VMEM_SHARED((NS * EPT,), jnp.int32),
        pltpu.VMEM_SHARED((NS * EPT,), jnp.int32),
        pltpu.VMEM_SHARED((N_PAD,), _f32),
    ],
    compiler_params=_lp_params,
)
def _sort_kernel(src_hbm, dst_hbm, osrc_hbm, odst_hbm, odeg_hbm,
                 src_v, dst_v, allc_v, cur_v, pos_v, shf, zbuf, ones_v,
                 cnt_sh, ssrc_sh, sdst_sh, deg_sh):
    c = lax.axis_index("c")
    s = lax.axis_index("s")
    erow = c * NS + s
    pltpu.sync_copy(src_hbm.at[erow], src_v)
    pltpu.sync_copy(dst_hbm.at[erow], dst_v)
    for k in range(40):
        zbuf[pl.ds(k * 16, 16)] = jnp.zeros((16,), _f32)
    for k in range(8):
        ones_v[pl.ds(k * 16, 16)] = jnp.ones((16,), _f32)
    pltpu.sync_copy(zbuf.at[pl.ds(0, N_PAD // NS)],
                    deg_sh.at[pl.ds(s * (N_PAD // NS), N_PAD // NS)])

    lane = lax.iota(jnp.int32, 16)
    zero = jnp.zeros((16,), jnp.int32)
    for b in range(NBKT):
        cur_v[pl.ds(b * 16, 16)] = zero
    shf[pl.ds(0, 16)] = zero

    def count_step(t, carry):
        sv = src_v[pl.ds(t * 16, 16)]
        slot = _bucket_slot(sv, lane)
        cur = plsc.load_gather(cur_v, [slot])
        plsc.store_scatter(cur_v, [slot], cur + 1)
        return carry

    lax.fori_loop(0, NV, count_step, 0)
    pltpu.sync_copy(cur_v, cnt_sh.at[s])
    plsc.subcore_barrier()
    pltpu.sync_copy(cnt_sh, allc_v)

    def lane_sum(x):
        for k in (1, 2, 4, 8):
            shf[pl.ds(8, 16)] = x
            x = x + shf[pl.ds(8 - k, 16)]
        return x

    gbase = jnp.int32(0)
    for b in range(NBKT):
        pre = zero
        allv = zero
        mine = zero
        for i in range(NS):
            v = allc_v[i, pl.ds(b * 16, 16)]
            pre = jnp.where(i < s, pre + v, pre)
            mine = jnp.where(i == s, v, mine)
            allv = allv + v
        pre_tiles = lane_sum(pre)[15]
        mine_in = lane_sum(mine)
        shf[pl.ds(8, 16)] = mine_in
        mine_excl = shf[pl.ds(7, 16)]
        cur_v[pl.ds(b * 16, 16)] = gbase + pre_tiles + mine_excl
        gbase = gbase + lane_sum(allv)[15]

    def place_step(t, carry):
        sv = src_v[pl.ds(t * 16, 16)]
        slot = _bucket_slot(sv, lane)
        pos = plsc.load_gather(cur_v, [slot])
        plsc.store_scatter(cur_v, [slot], pos + 1)
        pos_v[t // 8, pl.ds((t % 8) * 16, 16)] = pos
        return carry

    lax.fori_loop(0, NV, place_step, 0)

    def scat_step(j, carry):
        pltpu.sync_copy(src_v.at[pl.ds(j * CHUNK, CHUNK)], ssrc_sh.at[pos_v.at[j]])
        pltpu.sync_copy(dst_v.at[j], sdst_sh.at[pos_v.at[j]])
        pltpu.sync_copy(ones_v, deg_sh.at[dst_v.at[j]], add=True)
        return carry

    lax.fori_loop(0, EPT // CHUNK, scat_step, 0)
    plsc.subcore_barrier()
    pltpu.sync_copy(ssrc_sh.at[pl.ds(s * EPT, EPT)], osrc_hbm.at[c].at[pl.ds(s * EPT, EPT)])
    pltpu.sync_copy(sdst_sh.at[pl.ds(s * EPT, EPT)], odst_hbm.at[c].at[pl.ds(s * EPT, EPT)])
    RPT = N_PAD // NS
    pltpu.sync_copy(deg_sh.at[pl.ds(s * RPT, RPT)],
                    odeg_hbm.at[c].at[pl.ds(s * RPT, RPT)])


@functools.partial(
    pl.kernel,
    out_type=jax.ShapeDtypeStruct((NC, N_PAD, D), _f32),
    mesh=_mesh,
    scratch_types=[
        pltpu.VMEM((C // 2, CHUNK), jnp.int32),
        pltpu.VMEM((C // 2, CHUNK), jnp.int32),
        pltpu.VMEM((NB, CHUNK, D), _f32),
        pltpu.VMEM_SHARED((N_PAD, D), _f32),
        pltpu.SemaphoreType.DMA,
        pltpu.SemaphoreType.DMA,
    ],
)
def _agg_kernel(h_hbm, src_hbm, dst_hbm, out_hbm,
                src_v, dst_v, gbuf, acc_sh, sem0, sem1):
    c = lax.axis_index("c")
    s = lax.axis_index("s")
    wid = c * NS + s
    sems = [sem0, sem1]
    HC = C // 2

    def zrow(i, carry):
        for k in range(8):
            gbuf[0, i, pl.ds(k * 16, 16)] = jnp.zeros((16,), _f32)
        return carry

    lax.fori_loop(0, CHUNK, zrow, 0)
    for k in range(ROWS_PT // CHUNK):
        pltpu.sync_copy(gbuf.at[0], acc_sh.at[pl.ds(s * ROWS_PT + k * CHUNK, CHUNK)])
    plsc.subcore_barrier()

    for h in range(2):
        pltpu.sync_copy(src_hbm.at[wid].at[pl.ds(h * HC, HC)], src_v)
        pltpu.sync_copy(dst_hbm.at[wid].at[pl.ds(h * HC, HC)], dst_v)
        for b in range(NB):
            pltpu.async_copy(h_hbm.at[src_v.at[b]], gbuf.at[b], sems[b])

        def body(g, carry):
            base = g * NB
            for b in range(NB):
                j = base + b
                pltpu.make_async_copy(h_hbm.at[src_v.at[j]], gbuf.at[b], sems[b]).wait()
                pltpu.sync_copy(gbuf.at[b], acc_sh.at[dst_v.at[j]], add=True)
                nxt = j + NB

                @pl.when(nxt < HC)
                def _issue():
                    pltpu.async_copy(h_hbm.at[src_v.at[nxt]], gbuf.at[b], sems[b])

            return carry

        lax.fori_loop(0, HC // NB, body, 0)
    plsc.subcore_barrier()
    pltpu.sync_copy(
        acc_sh.at[pl.ds(s * ROWS_PT, ROWS_PT)],
        out_hbm.at[c].at[pl.ds(s * ROWS_PT, ROWS_PT)],
    )


_BR = 1000
_G = N // _BR


def _tc1_body(d_ref0, d_ref1, x_ref, w1_ref, ht_ref, dinv_ref):
    deg = d_ref0[0] + d_ref1[0] + 1.0
    dinv = lax.rsqrt(deg)
    h = jnp.dot(x_ref[...], w1_ref[...], preferred_element_type=_f32)
    ht_ref[...] = h * dinv
    dinv_ref[...] = dinv


def _tc1(degp, x, w1):
    return pl.pallas_call(
        _tc1_body,
        grid=(_G,),
        in_specs=[
            pl.BlockSpec((1, _BR, 1), lambda i: (0, i, 0)),
            pl.BlockSpec((1, _BR, 1), lambda i: (1, i, 0)),
            pl.BlockSpec((_BR, D), lambda i: (i, 0)),
            pl.BlockSpec((D, D), lambda i: (0, 0)),
        ],
        out_specs=[
            pl.BlockSpec((_BR, D), lambda i: (i, 0)),
            pl.BlockSpec((_BR, 1), lambda i: (i, 0)),
        ],
        out_shape=[
            jax.ShapeDtypeStruct((N, D), _f32),
            jax.ShapeDtypeStruct((N, 1), _f32),
        ],
    )(degp, degp, x, w1)


def _tc2_body(a_ref0, a_ref1, ht_ref, dinv_ref, b1_ref, w2_ref, ht2_ref):
    pre = (a_ref0[0] + a_ref1[0] + ht_ref[...]) * dinv_ref[...] + b1_ref[...]
    act = jnp.tanh(pre)
    h2 = jnp.dot(act, w2_ref[...], preferred_element_type=_f32)
    ht2_ref[...] = h2 * dinv_ref[...]


def _tc2(agg, ht, dinv, b1, w2):
    return pl.pallas_call(
        _tc2_body,
        grid=(_G,),
        in_specs=[
            pl.BlockSpec((1, _BR, D), lambda i: (0, i, 0)),
            pl.BlockSpec((1, _BR, D), lambda i: (1, i, 0)),
            pl.BlockSpec((_BR, D), lambda i: (i, 0)),
            pl.BlockSpec((_BR, 1), lambda i: (i, 0)),
            pl.BlockSpec((1, D), lambda i: (0, 0)),
            pl.BlockSpec((D, D), lambda i: (0, 0)),
        ],
        out_specs=pl.BlockSpec((_BR, D), lambda i: (i, 0)),
        out_shape=jax.ShapeDtypeStruct((N, D), _f32),
    )(agg, agg, ht, dinv, b1, w2)


def _tc3_body(a_ref0, a_ref1, ht2_ref, dinv_ref, b2_ref,
              wf1_ref, bf1_ref, wf2_ref, bf2_ref, out_ref):
    pre = (a_ref0[0] + a_ref1[0] + ht2_ref[...]) * dinv_ref[...] + b2_ref[...]
    act = jnp.tanh(pre)
    h3 = jnp.tanh(jnp.dot(act, wf1_ref[...], preferred_element_type=_f32)
                  + bf1_ref[...])
    out_ref[...] = jnp.dot(h3, wf2_ref[...], preferred_element_type=_f32) + bf2_ref[...]


def _tc3(agg, ht2, dinv, b2, wf1, bf1, wf2, bf2):
    return pl.pallas_call(
        _tc3_body,
        grid=(_G,),
        in_specs=[
            pl.BlockSpec((1, _BR, D), lambda i: (0, i, 0)),
            pl.BlockSpec((1, _BR, D), lambda i: (1, i, 0)),
            pl.BlockSpec((_BR, D), lambda i: (i, 0)),
            pl.BlockSpec((_BR, 1), lambda i: (i, 0)),
            pl.BlockSpec((1, D), lambda i: (0, 0)),
            pl.BlockSpec((D, 64), lambda i: (0, 0)),
            pl.BlockSpec((1, 64), lambda i: (0, 0)),
            pl.BlockSpec((64, 1), lambda i: (0, 0)),
            pl.BlockSpec((1, 1), lambda i: (0, 0)),
        ],
        out_specs=pl.BlockSpec((_BR, 1), lambda i: (i, 0)),
        out_shape=jax.ShapeDtypeStruct((N, 1), _f32),
    )(agg, agg, ht2, dinv, b2, wf1, bf1, wf2, bf2)


def kernel(x, edge_index, W1, b1, W2, b2, Wf1, bf1, Wf2, bf2):
    src = edge_index[0].astype(jnp.int32)
    dst = edge_index[1].astype(jnp.int32)
    pad = E_PAD - E
    ar = jnp.arange(pad, dtype=jnp.int32)
    dump = N + (ar % (N_PAD - N))
    src_p = jnp.concatenate([src, ar % N])
    dst_p = jnp.concatenate([dst, dump])
    src2 = src_p.reshape(NW, EPT)
    dst3 = dst_p.reshape(NW, C, CHUNK)

    srcg, dstg, degp = _sort_kernel(src2, dst3)
    srcg = srcg.reshape(NW, C, CHUNK)
    dstg = dstg.reshape(NW, C, CHUNK)
    degp = degp[:, :N].reshape(NC, N, 1)
    ht1, dinv = _tc1(degp, x, W1)
    agg1 = _agg_kernel(ht1, srcg, dstg)
    ht2 = _tc2(agg1, ht1, dinv, b1.reshape(1, D), W2)
    agg2 = _agg_kernel(ht2, srcg, dstg)
    return _tc3(agg2, ht2, dinv, b2.reshape(1, D),
                Wf1, bf1.reshape(1, 64), Wf2, bf2.reshape(1, 1))


# speedup vs baseline: 2.7804x; 1.0398x over previous
import functools

import jax
import jax.numpy as jnp
from jax import lax
from jax.experimental import pallas as pl
from jax.experimental.pallas import tpu as pltpu
from jax.experimental.pallas import tpu_sc as plsc

N = 10000
D = 128
E = 320000
NC = 2
NS = 16
NW = NC * NS
CHUNK = 128
C = 80
E_PAD = NW * C * CHUNK
N_PAD = 10240
ROWS_PT = N_PAD // NS
NB = 2

_mesh = plsc.VectorSubcoreMesh(core_axis_name="c", subcore_axis_name="s")
_f32 = jnp.float32


EPT = C * CHUNK
NBKT = 32
BKT_W = 320
NV = EPT // 16
_lp_params = pltpu.CompilerParams(needs_layout_passes=False)


def _bucket_slot(sv, lane):
    return lax.shift_left(sv // BKT_W, 4) + lane


@functools.partial(
    pl.kernel,
    out_type=[
        jax.ShapeDtypeStruct((NC, NS * EPT), jnp.int32),
        jax.ShapeDtypeStruct((NC, NS * EPT), jnp.int32),
        jax.ShapeDtypeStruct((NC, N_PAD), _f32),
    ],
    mesh=_mesh,
    scratch_types=[
        pltpu.VMEM((EPT,), jnp.int32),
        pltpu.VMEM((C, CHUNK), jnp.int32),
        pltpu.VMEM((NS, NBKT * 16), jnp.int32),
        pltpu.VMEM((NBKT * 16,), jnp.int32),
        pltpu.VMEM((EPT // CHUNK, CHUNK), jnp.int32),
        pltpu.VMEM((32,), jnp.int32),
        pltpu.VMEM((640,), _f32),
        pltpu.VMEM((CHUNK,), _f32),
        pltpu.VMEM_SHARED((NS, NBKT * 16), jnp.int32),
        pltpu.VMEM_SHARED((NS * EPT,), jnp.int32),
        pltpu.VMEM_SHARED((NS * EPT,), jnp.int32),
        pltpu.VMEM_SHARED((N_PAD,), _f32),
        pltpu.SemaphoreType.DMA,
        pltpu.SemaphoreType.DMA,
        pltpu.SemaphoreType.DMA,
    ],
    compiler_params=_lp_params,
)
def _sort_kernel(src_hbm, dst_hbm, osrc_hbm, odst_hbm, odeg_hbm,
                 src_v, dst_v, allc_v, cur_v, pos_v, shf, zbuf, ones_v,
                 cnt_sh, ssrc_sh, sdst_sh, deg_sh, sma, smb, smc):
    c = lax.axis_index("c")
    s = lax.axis_index("s")
    erow = c * NS + s
    pltpu.sync_copy(src_hbm.at[erow], src_v)
    pltpu.sync_copy(dst_hbm.at[erow], dst_v)
    for k in range(40):
        zbuf[pl.ds(k * 16, 16)] = jnp.zeros((16,), _f32)
    for k in range(8):
        ones_v[pl.ds(k * 16, 16)] = jnp.ones((16,), _f32)
    pltpu.sync_copy(zbuf.at[pl.ds(0, N_PAD // NS)],
                    deg_sh.at[pl.ds(s * (N_PAD // NS), N_PAD // NS)])

    lane = lax.iota(jnp.int32, 16)
    zero = jnp.zeros((16,), jnp.int32)
    for b in range(NBKT):
        cur_v[pl.ds(b * 16, 16)] = zero
    shf[pl.ds(0, 16)] = zero

    def count_step(t, carry):
        sv = src_v[pl.ds(t * 16, 16)]
        slot = _bucket_slot(sv, lane)
        cur = plsc.load_gather(cur_v, [slot])
        plsc.store_scatter(cur_v, [slot], cur + 1)
        return carry

    lax.fori_loop(0, NV, count_step, 0)
    pltpu.sync_copy(cur_v, cnt_sh.at[s])
    plsc.subcore_barrier()
    pltpu.sync_copy(cnt_sh, allc_v)

    def lane_sum(x):
        for k in (1, 2, 4, 8):
            shf[pl.ds(8, 16)] = x
            x = x + shf[pl.ds(8 - k, 16)]
        return x

    gbase = jnp.int32(0)
    for b in range(NBKT):
        pre = zero
        allv = zero
        mine = zero
        for i in range(NS):
            v = allc_v[i, pl.ds(b * 16, 16)]
            pre = jnp.where(i < s, pre + v, pre)
            mine = jnp.where(i == s, v, mine)
            allv = allv + v
        pre_tiles = lane_sum(pre)[15]
        mine_in = lane_sum(mine)
        shf[pl.ds(8, 16)] = mine_in
        mine_excl = shf[pl.ds(7, 16)]
        cur_v[pl.ds(b * 16, 16)] = gbase + pre_tiles + mine_excl
        gbase = gbase + lane_sum(allv)[15]

    def place_step(t, carry):
        sv = src_v[pl.ds(t * 16, 16)]
        slot = _bucket_slot(sv, lane)
        pos = plsc.load_gather(cur_v, [slot])
        plsc.store_scatter(cur_v, [slot], pos + 1)
        pos_v[t // 8, pl.ds((t % 8) * 16, 16)] = pos
        return carry

    lax.fori_loop(0, NV, place_step, 0)

    def scat_step(j, carry):
        pltpu.async_copy(src_v.at[pl.ds(j * CHUNK, CHUNK)], ssrc_sh.at[pos_v.at[j]], sma)
        pltpu.async_copy(dst_v.at[j], sdst_sh.at[pos_v.at[j]], smb)
        pltpu.async_copy(ones_v, deg_sh.at[dst_v.at[j]], smc, add=True)
        return carry

    lax.fori_loop(0, EPT // CHUNK, scat_step, 0)

    def drain_step(j, carry):
        pltpu.make_async_copy(src_v.at[pl.ds(0, CHUNK)], ssrc_sh.at[pos_v.at[0]], sma).wait()
        pltpu.make_async_copy(dst_v.at[0], sdst_sh.at[pos_v.at[0]], smb).wait()
        pltpu.make_async_copy(ones_v, deg_sh.at[dst_v.at[0]], smc).wait()
        return carry

    lax.fori_loop(0, EPT // CHUNK, drain_step, 0)
    plsc.subcore_barrier()
    pltpu.sync_copy(ssrc_sh.at[pl.ds(s * EPT, EPT)], osrc_hbm.at[c].at[pl.ds(s * EPT, EPT)])
    pltpu.sync_copy(sdst_sh.at[pl.ds(s * EPT, EPT)], odst_hbm.at[c].at[pl.ds(s * EPT, EPT)])
    RPT = N_PAD // NS
    pltpu.sync_copy(deg_sh.at[pl.ds(s * RPT, RPT)],
                    odeg_hbm.at[c].at[pl.ds(s * RPT, RPT)])


@functools.partial(
    pl.kernel,
    out_type=jax.ShapeDtypeStruct((NC, N_PAD, D), _f32),
    mesh=_mesh,
    scratch_types=[
        pltpu.VMEM((C // 2, CHUNK), jnp.int32),
        pltpu.VMEM((C // 2, CHUNK), jnp.int32),
        pltpu.VMEM((NB, CHUNK, D), _f32),
        pltpu.VMEM_SHARED((N_PAD, D), _f32),
        pltpu.SemaphoreType.DMA,
        pltpu.SemaphoreType.DMA,
    ],
)
def _agg_kernel(h_hbm, src_hbm, dst_hbm, out_hbm,
                src_v, dst_v, gbuf, acc_sh, sem0, sem1):
    c = lax.axis_index("c")
    s = lax.axis_index("s")
    wid = c * NS + s
    sems = [sem0, sem1]
    HC = C // 2

    def zrow(i, carry):
        for k in range(8):
            gbuf[0, i, pl.ds(k * 16, 16)] = jnp.zeros((16,), _f32)
        return carry

    lax.fori_loop(0, CHUNK, zrow, 0)
    for k in range(ROWS_PT // CHUNK):
        pltpu.sync_copy(gbuf.at[0], acc_sh.at[pl.ds(s * ROWS_PT + k * CHUNK, CHUNK)])
    plsc.subcore_barrier()

    for h in range(2):
        pltpu.sync_copy(src_hbm.at[wid].at[pl.ds(h * HC, HC)], src_v)
        pltpu.sync_copy(dst_hbm.at[wid].at[pl.ds(h * HC, HC)], dst_v)
        for b in range(NB):
            pltpu.async_copy(h_hbm.at[src_v.at[b]], gbuf.at[b], sems[b])

        def body(g, carry):
            base = g * NB
            for b in range(NB):
                j = base + b
                pltpu.make_async_copy(h_hbm.at[src_v.at[j]], gbuf.at[b], sems[b]).wait()
                pltpu.sync_copy(gbuf.at[b], acc_sh.at[dst_v.at[j]], add=True)
                nxt = j + NB

                @pl.when(nxt < HC)
                def _issue():
                    pltpu.async_copy(h_hbm.at[src_v.at[nxt]], gbuf.at[b], sems[b])

            return carry

        lax.fori_loop(0, HC // NB, body, 0)
    plsc.subcore_barrier()
    pltpu.sync_copy(
        acc_sh.at[pl.ds(s * ROWS_PT, ROWS_PT)],
        out_hbm.at[c].at[pl.ds(s * ROWS_PT, ROWS_PT)],
    )


_BR = 1000
_G = N // _BR


def _tc1_body(d_ref0, d_ref1, x_ref, w1_ref, ht_ref, dinv_ref):
    deg = d_ref0[0] + d_ref1[0] + 1.0
    dinv = lax.rsqrt(deg)
    h = jnp.dot(x_ref[...], w1_ref[...], preferred_element_type=_f32)
    ht_ref[...] = h * dinv
    dinv_ref[...] = dinv


def _tc1(degp, x, w1):
    return pl.pallas_call(
        _tc1_body,
        grid=(_G,),
        in_specs=[
            pl.BlockSpec((1, _BR, 1), lambda i: (0, i, 0)),
            pl.BlockSpec((1, _BR, 1), lambda i: (1, i, 0)),
            pl.BlockSpec((_BR, D), lambda i: (i, 0)),
            pl.BlockSpec((D, D), lambda i: (0, 0)),
        ],
        out_specs=[
            pl.BlockSpec((_BR, D), lambda i: (i, 0)),
            pl.BlockSpec((_BR, 1), lambda i: (i, 0)),
        ],
        out_shape=[
            jax.ShapeDtypeStruct((N, D), _f32),
            jax.ShapeDtypeStruct((N, 1), _f32),
        ],
    )(degp, degp, x, w1)


def _tc2_body(a_ref0, a_ref1, ht_ref, dinv_ref, b1_ref, w2_ref, ht2_ref):
    pre = (a_ref0[0] + a_ref1[0] + ht_ref[...]) * dinv_ref[...] + b1_ref[...]
    act = jnp.tanh(pre)
    h2 = jnp.dot(act, w2_ref[...], preferred_element_type=_f32)
    ht2_ref[...] = h2 * dinv_ref[...]


def _tc2(agg, ht, dinv, b1, w2):
    return pl.pallas_call(
        _tc2_body,
        grid=(_G,),
        in_specs=[
            pl.BlockSpec((1, _BR, D), lambda i: (0, i, 0)),
            pl.BlockSpec((1, _BR, D), lambda i: (1, i, 0)),
            pl.BlockSpec((_BR, D), lambda i: (i, 0)),
            pl.BlockSpec((_BR, 1), lambda i: (i, 0)),
            pl.BlockSpec((1, D), lambda i: (0, 0)),
            pl.BlockSpec((D, D), lambda i: (0, 0)),
        ],
        out_specs=pl.BlockSpec((_BR, D), lambda i: (i, 0)),
        out_shape=jax.ShapeDtypeStruct((N, D), _f32),
    )(agg, agg, ht, dinv, b1, w2)


def _tc3_body(a_ref0, a_ref1, ht2_ref, dinv_ref, b2_ref,
              wf1_ref, bf1_ref, wf2_ref, bf2_ref, out_ref):
    pre = (a_ref0[0] + a_ref1[0] + ht2_ref[...]) * dinv_ref[...] + b2_ref[...]
    act = jnp.tanh(pre)
    h3 = jnp.tanh(jnp.dot(act, wf1_ref[...], preferred_element_type=_f32)
                  + bf1_ref[...])
    out_ref[...] = jnp.dot(h3, wf2_ref[...], preferred_element_type=_f32) + bf2_ref[...]


def _tc3(agg, ht2, dinv, b2, wf1, bf1, wf2, bf2):
    return pl.pallas_call(
        _tc3_body,
        grid=(_G,),
        in_specs=[
            pl.BlockSpec((1, _BR, D), lambda i: (0, i, 0)),
            pl.BlockSpec((1, _BR, D), lambda i: (1, i, 0)),
            pl.BlockSpec((_BR, D), lambda i: (i, 0)),
            pl.BlockSpec((_BR, 1), lambda i: (i, 0)),
            pl.BlockSpec((1, D), lambda i: (0, 0)),
            pl.BlockSpec((D, 64), lambda i: (0, 0)),
            pl.BlockSpec((1, 64), lambda i: (0, 0)),
            pl.BlockSpec((64, 1), lambda i: (0, 0)),
            pl.BlockSpec((1, 1), lambda i: (0, 0)),
        ],
        out_specs=pl.BlockSpec((_BR, 1), lambda i: (i, 0)),
        out_shape=jax.ShapeDtypeStruct((N, 1), _f32),
    )(agg, agg, ht2, dinv, b2, wf1, bf1, wf2, bf2)


def kernel(x, edge_index, W1, b1, W2, b2, Wf1, bf1, Wf2, bf2):
    src = edge_index[0].astype(jnp.int32)
    dst = edge_index[1].astype(jnp.int32)
    pad = E_PAD - E
    ar = jnp.arange(pad, dtype=jnp.int32)
    dump = N + (ar % (N_PAD - N))
    src_p = jnp.concatenate([src, ar % N])
    dst_p = jnp.concatenate([dst, dump])
    src2 = src_p.reshape(NW, EPT)
    dst3 = dst_p.reshape(NW, C, CHUNK)

    srcg, dstg, degp = _sort_kernel(src2, dst3)
    srcg = srcg.reshape(NW, C, CHUNK)
    dstg = dstg.reshape(NW, C, CHUNK)
    degp = degp[:, :N].reshape(NC, N, 1)
    ht1, dinv = _tc1(degp, x, W1)
    agg1 = _agg_kernel(ht1, srcg, dstg)
    ht2 = _tc2(agg1, ht1, dinv, b1.reshape(1, D), W2)
    agg2 = _agg_kernel(ht2, srcg, dstg)
    return _tc3(agg2, ht2, dinv, b2.reshape(1, D),
                Wf1, bf1.reshape(1, 64), Wf2, bf2.reshape(1, 1))
